# baseline (speedup 1.0000x reference)
import functools

import jax
import jax.numpy as jnp
from jax.experimental import pallas as pl
from jax.experimental.pallas import tpu as pltpu
from jax.experimental.pallas import tpu_sc as plsc

BETA = 0.25
_BM = 512
_BN = 1024


def _dist_argmin_body(z_ref, cb_ref, zn_ref, cn_ref, idx_ref, dmin_ref):
    i = pl.program_id(0)
    j = pl.program_id(1)
    zt = z_ref[pl.ds(i * _BM, _BM), :]
    cb = cb_ref[pl.ds(j * _BN, _BN), :]
    dot = jax.lax.dot_general(
        zt, cb,
        dimension_numbers=(((1,), (1,)), ((), ())),
        preferred_element_type=jnp.float32,
    )
    zn = zn_ref[pl.ds(i * _BM, _BM), :]
    cn = cn_ref[:, pl.ds(j * _BN, _BN)]
    s = (zn + cn) - 2.0 * dot
    tile_min = jnp.min(s, axis=1, keepdims=True)
    cols = jax.lax.broadcasted_iota(jnp.int32, (_BM, _BN), 1)
    big = jnp.int32(2**31 - 1)
    local_arg = jnp.min(jnp.where(s == tile_min, cols, big), axis=1,
                        keepdims=True)
    tile_arg = local_arg + j * _BN

    row = pl.ds(i * _BM, _BM)

    @pl.when(j == 0)
    def _():
        dmin_ref[row, :] = tile_min
        idx_ref[row, :] = tile_arg

    @pl.when(j != 0)
    def _():
        prev = dmin_ref[row, :]
        better = tile_min < prev
        idx_ref[row, :] = jnp.where(better, tile_arg, idx_ref[row, :])
        dmin_ref[row, :] = jnp.minimum(tile_min, prev)


def _dist_argmin(z_flat, codebook, z_norm, c_norm, interpret=False):
    m, _ = z_flat.shape
    n, _ = codebook.shape
    grid = (m // _BM, n // _BN)
    full = lambda shape: pl.BlockSpec(shape, lambda i, j: tuple(0 for _ in shape))
    return pl.pallas_call(
        _dist_argmin_body,
        grid=grid,
        in_specs=[
            full(z_flat.shape),
            full(codebook.shape),
            full((m, 1)),
            full((1, n)),
        ],
        out_specs=[
            full((m, 1)),
            full((m, 1)),
        ],
        out_shape=[
            jax.ShapeDtypeStruct((m, 1), jnp.int32),
            jax.ShapeDtypeStruct((m, 1), jnp.float32),
        ],
        interpret=interpret,
    )(z_flat, codebook, z_norm, c_norm)


_GATHER_W = 128


def _sc_gather(codebook, idx_flat):
    num_idx = idx_flat.shape[0]
    e_dim = codebook.shape[1]
    idx2 = idx_flat.reshape(1, num_idx)
    mesh = plsc.VectorSubcoreMesh(core_axis_name="core",
                                  subcore_axis_name="subcore")

    @functools.partial(
        pl.kernel,
        out_type=jax.ShapeDtypeStruct((num_idx, e_dim), codebook.dtype),
        mesh=mesh,
    )
    def gather_kernel(cb_hbm, i_hbm, o_hbm):
        def body(i_vmem, o_vmem):
            pltpu.sync_copy(cb_hbm.at[i_vmem.at[0]], o_vmem)

        pltpu.emit_pipeline(
            body,
            grid=(num_idx // _GATHER_W,),
            in_specs=[pl.BlockSpec((1, _GATHER_W), index_map=lambda i: (0, i))],
            out_specs=[pl.BlockSpec((_GATHER_W, e_dim),
                                    index_map=lambda i: (i, 0))],
            core_axis_name=("core", "subcore"),
            dimension_semantics=(pltpu.PARALLEL,),
        )(i_hbm, o_hbm)

    return gather_kernel(codebook, idx2)


def kernel(z, codebook):
    zt = jnp.transpose(z, (0, 2, 3, 1))
    b, h, w, c = zt.shape
    z_flat = zt.reshape(-1, c)
    n_e = codebook.shape[0]
    z_norm = jnp.sum(z_flat ** 2, axis=1, keepdims=True)
    c_norm = jnp.sum(codebook ** 2, axis=1)[None, :]

    idx2, dmin = _dist_argmin(z_flat, codebook, z_norm, c_norm)
    idx_flat = idx2.reshape(-1)

    z_q_flat = _sc_gather(codebook, idx_flat)

    z_q = z_q_flat.reshape(zt.shape)
    mean_sq = jnp.sum(dmin) / (z_flat.shape[0] * c)
    loss = mean_sq + BETA * mean_sq
    z_q = zt + (z_q - zt)
    z_q = jnp.transpose(z_q, (0, 3, 1, 2))
    idx = idx_flat.reshape(b, h, w)
    return z_q, loss, idx
